# baseline (speedup 1.0000x reference)
import functools

import jax
import jax.numpy as jnp
from jax import lax
from jax.experimental import pallas as pl
from jax.experimental.pallas import tpu as pltpu
from jax.experimental.pallas import tpu_sc as plsc

_HIDDEN = 64

_NC = 2
_NS = 16
_NW = _NC * _NS

_RING = 4


def _gather_body(rows_per_w, seq, table_hbm, labels_hbm, out_hbm,
                 idx_v, rows, gsems, wsems):
    wid = lax.axis_index("s") * _NC + lax.axis_index("c")
    base = wid * rows_per_w
    nchunks = rows_per_w

    pltpu.sync_copy(labels_hbm.at[pl.ds(base * seq, rows_per_w * seq)], idx_v)

    def idx_slice(g):
        return idx_v.at[pl.ds(g * seq, seq)]

    def out_slice(g):
        return out_hbm.at[base + g]

    for r in range(_RING - 1):
        pltpu.async_copy(table_hbm.at[idx_slice(r)], rows[r], gsems[r])

    def step(j, carry):
        for b in range(_RING):
            g = _RING * j + b
            nb = (b + _RING - 1) % _RING

            @pl.when(g + _RING - 1 < nchunks)
            def _fire():
                @pl.when(g >= 1)
                def _drain():
                    pltpu.make_async_copy(
                        rows[nb], out_slice(g - 1), wsems[nb]).wait()
                pltpu.async_copy(
                    table_hbm.at[idx_slice(g + _RING - 1)],
                    rows[nb], gsems[nb])

            pltpu.make_async_copy(
                table_hbm.at[idx_slice(g)], rows[b], gsems[b]).wait()
            pltpu.async_copy(rows[b], out_slice(g), wsems[b])
        return carry

    lax.fori_loop(0, nchunks // _RING, step, 0)

    for r in range(_RING):
        g = nchunks - _RING + r
        pltpu.make_async_copy(rows[r], out_slice(g), wsems[r]).wait()


def kernel(pos_embedding_matrix, position_labels):
    b, s = position_labels.shape
    labels = position_labels.reshape(-1).astype(jnp.int32)
    assert b % (_NW * _RING) == 0
    rows_per_w = b // _NW

    def body(table_hbm, labels_hbm, out_hbm, idx_v,
             r0, r1, r2, r3, g0, g1, g2, g3, w0, w1, w2, w3):
        _gather_body(rows_per_w, s, table_hbm, labels_hbm, out_hbm,
                     idx_v, (r0, r1, r2, r3), (g0, g1, g2, g3),
                     (w0, w1, w2, w3))

    mesh = plsc.VectorSubcoreMesh(core_axis_name="c", subcore_axis_name="s")
    run = pl.kernel(
        body,
        mesh=mesh,
        compiler_params=pltpu.CompilerParams(use_tc_tiling_on_sc=False),
        out_type=jax.ShapeDtypeStruct((b, s, _HIDDEN), jnp.float32),
        scratch_types=(
            [pltpu.VMEM((rows_per_w * s,), jnp.int32)]
            + [pltpu.VMEM((s, _HIDDEN), jnp.float32)] * _RING
            + [pltpu.SemaphoreType.DMA] * (2 * _RING)
        ),
    )
    return run(pos_embedding_matrix, labels)
